# baseline (speedup 1.0000x reference)
import jax
import jax.numpy as jnp
from jax import lax
from jax.experimental import pallas as pl
from jax.experimental.pallas import tpu as pltpu

B_, P_, A_, D_ = 512, 256, 196, 128
NM, NU = 192, 64
BS = 16


def _enc_body(idx_ref, patches_ref, W_ref, b_ref, pos_ref, mtok_ref,
              ue_ref, me_ref, up_ref):
    W = W_ref[...]
    Wb = W.astype(jnp.bfloat16)
    bvec = b_ref[...]
    pos = pos_ref[...].astype(jnp.bfloat16)
    mvec = jnp.dot(mtok_ref[...], W, preferred_element_type=jnp.float32) + bvec
    for s in range(BS):
        idx_col = idx_ref[0, :, s:s + 1]
        D = (idx_col == lax.broadcasted_iota(jnp.int32, (P_, P_), 1)
             ).astype(jnp.bfloat16)
        du = D[NM:, :]
        dm = D[:NM, :]
        pb = patches_ref[s].astype(jnp.bfloat16)
        gp = jnp.dot(du, pb, preferred_element_type=jnp.float32
                     ).astype(jnp.bfloat16)
        upos = jnp.dot(du, pos, preferred_element_type=jnp.float32)
        mpos = jnp.dot(dm, pos, preferred_element_type=jnp.float32)
        ue_ref[s] = jnp.dot(gp, Wb, preferred_element_type=jnp.float32) + bvec + upos
        up_ref[s] = upos
        me_ref[s] = mvec + mpos


def _argsort_body(u_ref, it_ref):
    k = u_ref[...]
    ix = lax.broadcasted_iota(jnp.int32, (B_, P_), 1)
    pos = lax.broadcasted_iota(jnp.int32, (B_, P_), 1)
    for lvl in range(8):
        dirb = ((pos >> (lvl + 1)) & 1) == 1
        for j in range(lvl, -1, -1):
            d = 1 << j
            bitb = ((pos >> j) & 1) == 1
            pk = jnp.where(bitb, jnp.roll(k, d, axis=1),
                           jnp.roll(k, -d, axis=1))
            pi = jnp.where(bitb, jnp.roll(ix, d, axis=1),
                           jnp.roll(ix, -d, axis=1))
            less = (k < pk) | ((k == pk) & (ix < pi))
            keep = less ^ bitb ^ dirb
            k = jnp.where(keep, k, pk)
            ix = jnp.where(keep, ix, pi)
    it_ref[...] = ix


def kernel(patches, W, b, pos_table, mask_token, rand_uniform):
    idx_sorted = pl.pallas_call(
        _argsort_body,
        out_shape=jax.ShapeDtypeStruct((B_, P_), jnp.int32),
    )(rand_uniform)
    grid = (B_ // BS,)
    out_shapes = (
        jax.ShapeDtypeStruct((B_, NU, D_), jnp.float32),
        jax.ShapeDtypeStruct((B_, NM, D_), jnp.float32),
        jax.ShapeDtypeStruct((B_, NU, D_), jnp.float32),
    )
    ue, me, up = pl.pallas_call(
        _enc_body,
        grid=grid,
        in_specs=[
            pl.BlockSpec((1, P_, BS), lambda i: (i, 0, 0)),
            pl.BlockSpec((BS, P_, A_), lambda i: (i, 0, 0)),
            pl.BlockSpec((A_, D_), lambda i: (0, 0)),
            pl.BlockSpec((1, D_), lambda i: (0, 0)),
            pl.BlockSpec((P_, D_), lambda i: (0, 0)),
            pl.BlockSpec((1, A_), lambda i: (0, 0)),
        ],
        out_specs=(
            pl.BlockSpec((BS, NU, D_), lambda i: (i, 0, 0)),
            pl.BlockSpec((BS, NM, D_), lambda i: (i, 0, 0)),
            pl.BlockSpec((BS, NU, D_), lambda i: (i, 0, 0)),
        ),
        out_shape=out_shapes,
        compiler_params=pltpu.CompilerParams(
            dimension_semantics=("parallel",),
        ),
    )(idx_sorted.reshape(B_ // BS, BS, P_).swapaxes(1, 2), patches, W,
      b.reshape(1, D_), pos_table, mask_token)
    mask_indices = idx_sorted[:, :NM]
    unmask_indices = idx_sorted[:, NM:]
    return ue, me, up, mask_indices, unmask_indices
